# baseline (speedup 1.0000x reference)
import functools

import numpy as np
import jax
import jax.numpy as jnp
from jax import lax
from jax.experimental import pallas as pl
from jax.experimental.pallas import tpu as pltpu
from jax.experimental.pallas import tpu_sc as plsc


def _np_rotl(x, d):
    d = np.uint32(d)
    return (x << d) | (x >> np.uint32(32 - d))


def _np_threefry2x32(k0, k1, c1, c2):
    rot = ((13, 15, 26, 6), (17, 29, 16, 24))
    ks0 = np.uint32(k0)
    ks1 = np.uint32(k1)
    ks2 = ks0 ^ ks1 ^ np.uint32(0x1BD11BDA)
    ks = (ks0, ks1, ks2)
    x0 = (c1 + ks0).astype(np.uint32)
    x1 = (c2 + ks1).astype(np.uint32)
    for i in range(5):
        for r in rot[i % 2]:
            x0 = (x0 + x1).astype(np.uint32)
            x1 = _np_rotl(x1, r)
            x1 = x0 ^ x1
        x0 = (x0 + ks[(i + 1) % 3]).astype(np.uint32)
        x1 = (x1 + ks[(i + 2) % 3] + np.uint32(i + 1)).astype(np.uint32)
    return x0, x1


def _np_split(key, num):
    idx = np.arange(num, dtype=np.uint64)
    c1 = (idx >> np.uint64(32)).astype(np.uint32)
    c2 = (idx & np.uint64(0xFFFFFFFF)).astype(np.uint32)
    b1, b2 = _np_threefry2x32(key[0], key[1], c1, c2)
    return [(b1[i], b2[i]) for i in range(num)]

_KEY = (np.uint32(0), np.uint32(42))
_K1, _K2, _K3 = _np_split(_KEY, 3)
_KA = _np_split(_K1, 2)[1]


def _rotl(x, d):
    return lax.shift_left(x, np.uint32(d)) | lax.shift_right_logical(
        x, np.uint32(32 - d))


def _threefry(ks0, ks1, c2):
    rot = ((13, 15, 26, 6), (17, 29, 16, 24))
    ks2 = ks0 ^ ks1 ^ np.uint32(0x1BD11BDA)
    ks = (ks0, ks1, ks2)
    x0 = ks0 + jnp.zeros_like(c2)
    x1 = c2 + ks1
    for i in range(5):
        for r in rot[i % 2]:
            x0 = x0 + x1
            x1 = _rotl(x1, r)
            x1 = x0 ^ x1
        x0 = x0 + ks[(i + 1) % 3]
        x1 = x1 + ks[(i + 2) % 3] + np.uint32(i + 1)
    return x0 ^ x1


def _gather_q(q_params, labels):
    B = labels.shape[0]
    H = q_params.shape[1]
    info = plsc.get_sparse_core_info()
    nw = info.num_cores * info.num_subcores
    b_per_w = B // nw
    mesh = plsc.VectorSubcoreMesh(core_axis_name="c", subcore_axis_name="s")

    @functools.partial(
        pl.kernel,
        mesh=mesh,
        out_type=jax.ShapeDtypeStruct((B, H), jnp.float32),
        scratch_types=[
            pltpu.VMEM((b_per_w,), jnp.int32),
            pltpu.VMEM((b_per_w, H), jnp.float32),
            pltpu.SemaphoreType.DMA,
        ],
    )
    def k(table_hbm, idx_hbm, out_hbm, idx_v, rows_v, sem):
        wid = lax.axis_index("s") * info.num_cores + lax.axis_index("c")
        base = wid * b_per_w
        pltpu.sync_copy(idx_hbm.at[pl.ds(base, b_per_w)], idx_v)
        pltpu.async_copy(table_hbm.at[idx_v], rows_v, sem).wait()
        pltpu.sync_copy(rows_v, out_hbm.at[pl.ds(base, b_per_w)])

    return k(q_params, labels)


def _sample_body(num_ops, num_scales, max_t, bm,
                 q_ref, w_ref, augout_ref, scout_ref):
    win = num_ops * 32
    nts = max_t * num_scales
    g2base = max_t * 32
    augbase = g2base + max_t

    i = pl.program_id(0)
    rowg = lax.broadcasted_iota(jnp.uint32, (bm, 128), 0) + np.uint32(bm) * i.astype(jnp.uint32)
    col = lax.broadcasted_iota(jnp.uint32, (bm, 128), 1)
    is_g3 = col < np.uint32(g2base)
    is_g2 = col < np.uint32(augbase)
    counts = jnp.where(
        is_g3,
        np.uint32(nts) * rowg + col - lax.shift_right_logical(col, np.uint32(5)),
        jnp.where(is_g2, np.uint32(max_t) * rowg + col - np.uint32(g2base),
                  np.uint32(max_t) * rowg + col - np.uint32(augbase)))
    k0 = jnp.where(is_g3, np.uint32(_K3[0]),
                   jnp.where(is_g2, np.uint32(_K2[0]), np.uint32(_KA[0])))
    k1 = jnp.where(is_g3, np.uint32(_K3[1]),
                   jnp.where(is_g2, np.uint32(_K2[1]), np.uint32(_KA[1])))
    bits = _threefry(k0, k1, counts)

    fb = lax.shift_right_logical(bits, np.uint32(9)) | np.uint32(0x3F800000)
    f = lax.bitcast_convert_type(fb, jnp.float32) - 1.0
    tiny = np.float32(np.finfo(np.float32).tiny)
    u = jnp.maximum(tiny, f * np.float32(1.0) + tiny)
    gum = -jnp.log(-jnp.log(u))
    augbits = lax.bitcast_convert_type(bits & np.uint32(15), jnp.int32)

    acc = lax.dot_general(q_ref[...], w_ref[...],
                          (((1,), (1,)), ((), ())),
                          preferred_element_type=jnp.float32)

    a0 = acc[:, win:win + 1] + gum[:, g2base:g2base + 1]
    a1 = acc[:, win + 1:win + 2] + gum[:, g2base + 1:g2base + 2]
    a2 = acc[:, win + 2:win + 3] + gum[:, g2base + 2:g2base + 3]
    nt_idx = jnp.where(a1 > a0, 1, 0)
    nt_idx = jnp.where(a2 > jnp.maximum(a0, a1), 2, nt_idx)
    n_transforms = nt_idx + 1

    slices = [acc[:, o * 32:(o + 1) * 32] for o in range(num_ops)]
    col32 = lax.broadcasted_iota(jnp.int32, (bm, 32), 1)
    is_pad = col32 >= num_scales
    aug_cols = []
    sc_cols = []
    for t in range(max_t):
        ind_t = augbits[:, augbase + t:augbase + t + 1]
        folded = slices[0]
        for o in range(1, num_ops):
            folded = jnp.where(ind_t == o, slices[o], folded)
        noise_t = jnp.where(is_pad, -1e30,
                            gum[:, t * 32:(t + 1) * 32])
        vals = folded + noise_t
        mx = jnp.max(vals, axis=1, keepdims=True)
        scale_t = jnp.min(jnp.where(vals == mx, col32, 32), axis=1,
                          keepdims=True)
        sc_cols.append(scale_t)
        aug_cols.append(jnp.where(t < n_transforms, ind_t, 0))
    augout_ref[...] = jnp.concatenate(aug_cols, axis=1)
    scout_ref[...] = jnp.concatenate(sc_cols, axis=1)


def kernel(imgs, labels, q_params, op_embs, num_transforms_embs, scale_embs):
    B = imgs.shape[0]
    num_ops, num_scales, H = scale_embs.shape
    max_t = num_transforms_embs.shape[0]
    w = jnp.concatenate(
        [jnp.pad(scale_embs, ((0, 0), (0, 32 - num_scales), (0, 0))
                 ).reshape(num_ops * 32, H),
         num_transforms_embs], axis=0)

    q = _gather_q(q_params, labels.astype(jnp.int32))

    bm = 512
    grid = (B // bm,)
    wrows = num_ops * 32 + max_t
    augout, scout = pl.pallas_call(
        functools.partial(_sample_body, num_ops, num_scales, max_t, bm),
        grid=grid,
        in_specs=[
            pl.BlockSpec((bm, H), lambda i: (i, 0)),
            pl.BlockSpec((wrows, H), lambda i: (0, 0)),
        ],
        out_specs=[
            pl.BlockSpec((bm, max_t), lambda i: (i, 0)),
            pl.BlockSpec((bm, max_t), lambda i: (i, 0)),
        ],
        out_shape=[
            jax.ShapeDtypeStruct((B, max_t), jnp.int32),
            jax.ShapeDtypeStruct((B, max_t), jnp.int32),
        ],
    )(q, w)

    return (augout, scout)


# speedup vs baseline: 1.6690x; 1.6690x over previous
import functools

import numpy as np
import jax
import jax.numpy as jnp
from jax import lax
from jax.experimental import pallas as pl
from jax.experimental.pallas import tpu as pltpu
from jax.experimental.pallas import tpu_sc as plsc


def _np_rotl(x, d):
    d = np.uint32(d)
    return (x << d) | (x >> np.uint32(32 - d))


def _np_threefry2x32(k0, k1, c1, c2):
    rot = ((13, 15, 26, 6), (17, 29, 16, 24))
    ks0 = np.uint32(k0)
    ks1 = np.uint32(k1)
    ks2 = ks0 ^ ks1 ^ np.uint32(0x1BD11BDA)
    ks = (ks0, ks1, ks2)
    x0 = (c1 + ks0).astype(np.uint32)
    x1 = (c2 + ks1).astype(np.uint32)
    for i in range(5):
        for r in rot[i % 2]:
            x0 = (x0 + x1).astype(np.uint32)
            x1 = _np_rotl(x1, r)
            x1 = x0 ^ x1
        x0 = (x0 + ks[(i + 1) % 3]).astype(np.uint32)
        x1 = (x1 + ks[(i + 2) % 3] + np.uint32(i + 1)).astype(np.uint32)
    return x0, x1


def _np_split(key, num):
    idx = np.arange(num, dtype=np.uint64)
    c1 = (idx >> np.uint64(32)).astype(np.uint32)
    c2 = (idx & np.uint64(0xFFFFFFFF)).astype(np.uint32)
    b1, b2 = _np_threefry2x32(key[0], key[1], c1, c2)
    return [(b1[i], b2[i]) for i in range(num)]

_KEY = (np.uint32(0), np.uint32(42))
_K1, _K2, _K3 = _np_split(_KEY, 3)
_KA = _np_split(_K1, 2)[1]


def _rotl(x, d):
    return lax.shift_left(x, np.uint32(d)) | lax.shift_right_logical(
        x, np.uint32(32 - d))


def _threefry(ks0, ks1, c2):
    rot = ((13, 15, 26, 6), (17, 29, 16, 24))
    ks2 = ks0 ^ ks1 ^ np.uint32(0x1BD11BDA)
    ks = (ks0, ks1, ks2)
    x0 = ks0 + jnp.zeros_like(c2)
    x1 = c2 + ks1
    for i in range(5):
        for r in rot[i % 2]:
            x0 = x0 + x1
            x1 = _rotl(x1, r)
            x1 = x0 ^ x1
        x0 = x0 + ks[(i + 1) % 3]
        x1 = x1 + ks[(i + 2) % 3] + np.uint32(i + 1)
    return x0 ^ x1


def _gather_q(q_params, labels):
    B = labels.shape[0]
    H = q_params.shape[1]
    info = plsc.get_sparse_core_info()
    nw = info.num_cores * info.num_subcores
    b_per_w = B // nw
    mesh = plsc.VectorSubcoreMesh(core_axis_name="c", subcore_axis_name="s")

    @functools.partial(
        pl.kernel,
        mesh=mesh,
        out_type=jax.ShapeDtypeStruct((B, H), jnp.float32),
        scratch_types=[
            pltpu.VMEM((b_per_w,), jnp.int32),
            pltpu.VMEM((b_per_w, H), jnp.float32),
            pltpu.SemaphoreType.DMA,
        ],
    )
    def k(table_hbm, idx_hbm, out_hbm, idx_v, rows_v, sem):
        wid = lax.axis_index("s") * info.num_cores + lax.axis_index("c")
        base = wid * b_per_w
        pltpu.sync_copy(idx_hbm.at[pl.ds(base, b_per_w)], idx_v)
        pltpu.async_copy(table_hbm.at[idx_v], rows_v, sem).wait()
        pltpu.sync_copy(rows_v, out_hbm.at[pl.ds(base, b_per_w)])

    return k(q_params, labels)


def _sample_body(num_ops, num_scales, max_t, bm,
                 q_ref, se_ref, nte_ref, augout_ref, scout_ref):
    nts = max_t * num_scales
    g2base = nts
    augbase = nts + max_t
    NEG = np.float32(-1e30)

    i = pl.program_id(0)
    rowg = lax.broadcasted_iota(jnp.uint32, (bm, 128), 0) + np.uint32(bm) * i.astype(jnp.uint32)
    col = lax.broadcasted_iota(jnp.uint32, (bm, 128), 1)
    is_g3 = col < np.uint32(g2base)
    is_g2 = col < np.uint32(augbase)
    counts = jnp.where(
        is_g3, np.uint32(nts) * rowg + col,
        np.uint32(max_t) * rowg + col -
        jnp.where(is_g2, np.uint32(g2base), np.uint32(augbase)))
    k0 = jnp.where(is_g3, np.uint32(_K3[0]),
                   jnp.where(is_g2, np.uint32(_K2[0]), np.uint32(_KA[0])))
    k1 = jnp.where(is_g3, np.uint32(_K3[1]),
                   jnp.where(is_g2, np.uint32(_K2[1]), np.uint32(_KA[1])))
    bits = _threefry(k0, k1, counts)

    fb = lax.shift_right_logical(bits, np.uint32(9)) | np.uint32(0x3F800000)
    f = lax.bitcast_convert_type(fb, jnp.float32) - 1.0
    tiny = np.float32(np.finfo(np.float32).tiny)
    u = jnp.maximum(tiny, f * np.float32(1.0) + tiny)
    gum = -jnp.log(-jnp.log(u))
    augbits = lax.bitcast_convert_type(bits & np.uint32(15), jnp.int32)

    se = se_ref[...]
    w = jnp.concatenate(
        [se, jnp.zeros((num_ops, 32 - num_scales, 128), jnp.float32)],
        axis=1).reshape(num_ops * 32, 128)
    acc = lax.dot_general(q_ref[...], w, (((1,), (1,)), ((), ())),
                          preferred_element_type=jnp.float32)
    ant = lax.dot_general(q_ref[...], nte_ref[...], (((1,), (1,)), ((), ())),
                          preferred_element_type=jnp.float32)

    a0 = ant[:, 0:1] + gum[:, g2base:g2base + 1]
    a1 = ant[:, 1:2] + gum[:, g2base + 1:g2base + 2]
    a2 = ant[:, 2:3] + gum[:, g2base + 2:g2base + 3]
    nt_idx = jnp.where(a1 > a0, 1, 0)
    nt_idx = jnp.where(a2 > jnp.maximum(a0, a1), 2, nt_idx)
    n_transforms = nt_idx + 1

    quarters = [acc[:, j * 128:(j + 1) * 128] for j in range(4)]
    col128 = lax.broadcasted_iota(jnp.int32, (bm, 128), 1)
    aug_cols = []
    sc_cols = []
    for t in range(max_t):
        ind_t = augbits[:, augbase + t:augbase + t + 1]
        j0 = lax.shift_right_logical(ind_t, 2)
        lb = ind_t & 3
        fq = jnp.where(j0 == 0, quarters[0],
                       jnp.where(j0 == 1, quarters[1],
                                 jnp.where(j0 == 2, quarters[2], quarters[3])))
        m = jnp.roll(gum, -num_scales * t, axis=1) if t else gum
        m = jnp.where(col128 < num_scales, m, NEG)
        m = jnp.where(col128 >= 32, jnp.roll(m, 32, axis=1), m)
        m = jnp.where(col128 >= 64, jnp.roll(m, 64, axis=1), m)
        vals = jnp.where(lax.shift_right_logical(col128, 5) == lb,
                         fq + m, NEG)
        mx = jnp.max(vals, axis=1, keepdims=True)
        sc = jnp.min(jnp.where(vals == mx, col128, 128), axis=1,
                     keepdims=True)
        sc_cols.append(sc & 31)
        aug_cols.append(jnp.where(t < n_transforms, ind_t, 0))
    augout_ref[...] = jnp.concatenate(aug_cols, axis=1)
    scout_ref[...] = jnp.concatenate(sc_cols, axis=1)


def kernel(imgs, labels, q_params, op_embs, num_transforms_embs, scale_embs):
    B = imgs.shape[0]
    num_ops, num_scales, H = scale_embs.shape
    max_t = num_transforms_embs.shape[0]
    q = _gather_q(q_params, labels.astype(jnp.int32))

    bm = 512
    grid = (B // bm,)
    augout, scout = pl.pallas_call(
        functools.partial(_sample_body, num_ops, num_scales, max_t, bm),
        grid=grid,
        in_specs=[
            pl.BlockSpec((bm, H), lambda i: (i, 0)),
            pl.BlockSpec((num_ops, num_scales, H), lambda i: (0, 0, 0)),
            pl.BlockSpec((max_t, H), lambda i: (0, 0)),
        ],
        out_specs=[
            pl.BlockSpec((bm, max_t), lambda i: (i, 0)),
            pl.BlockSpec((bm, max_t), lambda i: (i, 0)),
        ],
        out_shape=[
            jax.ShapeDtypeStruct((B, max_t), jnp.int32),
            jax.ShapeDtypeStruct((B, max_t), jnp.int32),
        ],
    )(q, scale_embs, num_transforms_embs)

    return (augout, scout)


# speedup vs baseline: 1.8962x; 1.1361x over previous
import functools

import numpy as np
import jax
import jax.numpy as jnp
from jax import lax
from jax.experimental import pallas as pl
from jax.experimental.pallas import tpu as pltpu
from jax.experimental.pallas import tpu_sc as plsc


def _np_rotl(x, d):
    d = np.uint32(d)
    return (x << d) | (x >> np.uint32(32 - d))


def _np_threefry2x32(k0, k1, c1, c2):
    rot = ((13, 15, 26, 6), (17, 29, 16, 24))
    ks0 = np.uint32(k0)
    ks1 = np.uint32(k1)
    ks2 = ks0 ^ ks1 ^ np.uint32(0x1BD11BDA)
    ks = (ks0, ks1, ks2)
    x0 = (c1 + ks0).astype(np.uint32)
    x1 = (c2 + ks1).astype(np.uint32)
    for i in range(5):
        for r in rot[i % 2]:
            x0 = (x0 + x1).astype(np.uint32)
            x1 = _np_rotl(x1, r)
            x1 = x0 ^ x1
        x0 = (x0 + ks[(i + 1) % 3]).astype(np.uint32)
        x1 = (x1 + ks[(i + 2) % 3] + np.uint32(i + 1)).astype(np.uint32)
    return x0, x1


def _np_split(key, num):
    idx = np.arange(num, dtype=np.uint64)
    c1 = (idx >> np.uint64(32)).astype(np.uint32)
    c2 = (idx & np.uint64(0xFFFFFFFF)).astype(np.uint32)
    b1, b2 = _np_threefry2x32(key[0], key[1], c1, c2)
    return [(b1[i], b2[i]) for i in range(num)]

_KEY = (np.uint32(0), np.uint32(42))
_K1, _K2, _K3 = _np_split(_KEY, 3)
_KA = _np_split(_K1, 2)[1]


def _rotl(x, d):
    return lax.shift_left(x, np.uint32(d)) | lax.shift_right_logical(
        x, np.uint32(32 - d))


def _threefry(ks0, ks1, c2):
    rot = ((13, 15, 26, 6), (17, 29, 16, 24))
    ks2 = ks0 ^ ks1 ^ np.uint32(0x1BD11BDA)
    ks = (ks0, ks1, ks2)
    x0 = ks0 + jnp.zeros_like(c2)
    x1 = c2 + ks1
    for i in range(5):
        for r in rot[i % 2]:
            x0 = x0 + x1
            x1 = _rotl(x1, r)
            x1 = x0 ^ x1
        x0 = x0 + ks[(i + 1) % 3]
        x1 = x1 + ks[(i + 2) % 3] + np.uint32(i + 1)
    return x0 ^ x1


def _gather_q(q_params, labels):
    B = labels.shape[0]
    H = q_params.shape[1]
    info = plsc.get_sparse_core_info()
    nw = info.num_cores * info.num_subcores
    b_per_w = B // nw
    mesh = plsc.VectorSubcoreMesh(core_axis_name="c", subcore_axis_name="s")

    @functools.partial(
        pl.kernel,
        mesh=mesh,
        out_type=jax.ShapeDtypeStruct((B, H), jnp.float32),
        scratch_types=[
            pltpu.VMEM((b_per_w,), jnp.int32),
            pltpu.VMEM((b_per_w, H), jnp.float32),
            pltpu.SemaphoreType.DMA,
        ],
    )
    def k(table_hbm, idx_hbm, out_hbm, idx_v, rows_v, sem):
        wid = lax.axis_index("s") * info.num_cores + lax.axis_index("c")
        base = wid * b_per_w
        pltpu.sync_copy(idx_hbm.at[pl.ds(base, b_per_w)], idx_v)
        pltpu.async_copy(table_hbm.at[idx_v], rows_v, sem).wait()
        pltpu.sync_copy(rows_v, out_hbm.at[pl.ds(base, b_per_w)])

    return k(q_params, labels)


def _sample_body(num_ops, num_scales, max_t, bm,
                 lab_ref, qp_ref, se_ref, nte_ref, augout_ref, scout_ref):
    nts = max_t * num_scales
    g2base = nts
    augbase = nts + max_t
    NEG = np.float32(-1e30)

    i = pl.program_id(0)
    rowg = lax.broadcasted_iota(jnp.uint32, (bm, 128), 0) + np.uint32(bm) * i.astype(jnp.uint32)
    col = lax.broadcasted_iota(jnp.uint32, (bm, 128), 1)
    is_g3 = col < np.uint32(g2base)
    is_g2 = col < np.uint32(augbase)
    counts = jnp.where(
        is_g3, np.uint32(nts) * rowg + col,
        np.uint32(max_t) * rowg + col -
        jnp.where(is_g2, np.uint32(g2base), np.uint32(augbase)))
    k0 = jnp.where(is_g3, np.uint32(_K3[0]),
                   jnp.where(is_g2, np.uint32(_K2[0]), np.uint32(_KA[0])))
    k1 = jnp.where(is_g3, np.uint32(_K3[1]),
                   jnp.where(is_g2, np.uint32(_K2[1]), np.uint32(_KA[1])))
    bits = _threefry(k0, k1, counts)

    fb = lax.shift_right_logical(bits, np.uint32(9)) | np.uint32(0x3F800000)
    f = lax.bitcast_convert_type(fb, jnp.float32) - 1.0
    tiny = np.float32(np.finfo(np.float32).tiny)
    u = jnp.maximum(tiny, f * np.float32(1.0) + tiny)
    gum = -jnp.log(-jnp.log(u))
    augbits = lax.bitcast_convert_type(bits & np.uint32(15), jnp.int32)

    nl = qp_ref.shape[0]
    lab = lab_ref[...]
    colv = lax.broadcasted_iota(jnp.int32, (bm, nl), 1)
    onehot = jnp.where(colv == lab, 1.0, 0.0).astype(jnp.float32)
    q = lax.dot_general(onehot, qp_ref[...], (((1,), (0,)), ((), ())),
                        preferred_element_type=jnp.float32)

    se = se_ref[...]
    w = jnp.concatenate(
        [se, jnp.zeros((num_ops, 32 - num_scales, 128), jnp.float32)],
        axis=1).reshape(num_ops * 32, 128)
    acc = lax.dot_general(q, w, (((1,), (1,)), ((), ())),
                          preferred_element_type=jnp.float32)
    ant = lax.dot_general(q, nte_ref[...], (((1,), (1,)), ((), ())),
                          preferred_element_type=jnp.float32)

    a0 = ant[:, 0:1] + gum[:, g2base:g2base + 1]
    a1 = ant[:, 1:2] + gum[:, g2base + 1:g2base + 2]
    a2 = ant[:, 2:3] + gum[:, g2base + 2:g2base + 3]
    nt_idx = jnp.where(a1 > a0, 1, 0)
    nt_idx = jnp.where(a2 > jnp.maximum(a0, a1), 2, nt_idx)
    n_transforms = nt_idx + 1

    quarters = [acc[:, j * 128:(j + 1) * 128] for j in range(4)]
    col128 = lax.broadcasted_iota(jnp.int32, (bm, 128), 1)
    aug_cols = []
    sc_cols = []
    for t in range(max_t):
        ind_t = augbits[:, augbase + t:augbase + t + 1]
        j0 = lax.shift_right_logical(ind_t, 2)
        lb = ind_t & 3
        fq = jnp.where(j0 == 0, quarters[0],
                       jnp.where(j0 == 1, quarters[1],
                                 jnp.where(j0 == 2, quarters[2], quarters[3])))
        m = jnp.roll(gum, -num_scales * t, axis=1) if t else gum
        m = jnp.where(col128 < num_scales, m, NEG)
        m = jnp.where(col128 >= 32, jnp.roll(m, 32, axis=1), m)
        m = jnp.where(col128 >= 64, jnp.roll(m, 64, axis=1), m)
        vals = jnp.where(lax.shift_right_logical(col128, 5) == lb,
                         fq + m, NEG)
        mx = jnp.max(vals, axis=1, keepdims=True)
        sc = jnp.min(jnp.where(vals == mx, col128, 128), axis=1,
                     keepdims=True)
        sc_cols.append(sc & 31)
        aug_cols.append(jnp.where(t < n_transforms, ind_t, 0))
    augout_ref[...] = jnp.concatenate(aug_cols, axis=1)
    scout_ref[...] = jnp.concatenate(sc_cols, axis=1)


def kernel(imgs, labels, q_params, op_embs, num_transforms_embs, scale_embs):
    B = imgs.shape[0]
    num_ops, num_scales, H = scale_embs.shape
    max_t = num_transforms_embs.shape[0]
    nl = q_params.shape[0]
    lab2 = labels.astype(jnp.int32).reshape(B, 1)

    bm = 512
    grid = (B // bm,)
    augout, scout = pl.pallas_call(
        functools.partial(_sample_body, num_ops, num_scales, max_t, bm),
        grid=grid,
        in_specs=[
            pl.BlockSpec((bm, 1), lambda i: (i, 0)),
            pl.BlockSpec((nl, H), lambda i: (0, 0)),
            pl.BlockSpec((num_ops, num_scales, H), lambda i: (0, 0, 0)),
            pl.BlockSpec((max_t, H), lambda i: (0, 0)),
        ],
        out_specs=[
            pl.BlockSpec((bm, max_t), lambda i: (i, 0)),
            pl.BlockSpec((bm, max_t), lambda i: (i, 0)),
        ],
        out_shape=[
            jax.ShapeDtypeStruct((B, max_t), jnp.int32),
            jax.ShapeDtypeStruct((B, max_t), jnp.int32),
        ],
    )(lab2, q_params, scale_embs, num_transforms_embs)

    return (augout, scout)


# speedup vs baseline: 2.3791x; 1.2547x over previous
import functools

import numpy as np
import jax
import jax.numpy as jnp
from jax import lax
from jax.experimental import pallas as pl
from jax.experimental.pallas import tpu as pltpu
from jax.experimental.pallas import tpu_sc as plsc


def _np_rotl(x, d):
    d = np.uint32(d)
    return (x << d) | (x >> np.uint32(32 - d))


def _np_threefry2x32(k0, k1, c1, c2):
    rot = ((13, 15, 26, 6), (17, 29, 16, 24))
    ks0 = np.uint32(k0)
    ks1 = np.uint32(k1)
    ks2 = ks0 ^ ks1 ^ np.uint32(0x1BD11BDA)
    ks = (ks0, ks1, ks2)
    x0 = (c1 + ks0).astype(np.uint32)
    x1 = (c2 + ks1).astype(np.uint32)
    for i in range(5):
        for r in rot[i % 2]:
            x0 = (x0 + x1).astype(np.uint32)
            x1 = _np_rotl(x1, r)
            x1 = x0 ^ x1
        x0 = (x0 + ks[(i + 1) % 3]).astype(np.uint32)
        x1 = (x1 + ks[(i + 2) % 3] + np.uint32(i + 1)).astype(np.uint32)
    return x0, x1


def _np_split(key, num):
    idx = np.arange(num, dtype=np.uint64)
    c1 = (idx >> np.uint64(32)).astype(np.uint32)
    c2 = (idx & np.uint64(0xFFFFFFFF)).astype(np.uint32)
    b1, b2 = _np_threefry2x32(key[0], key[1], c1, c2)
    return [(b1[i], b2[i]) for i in range(num)]

_KEY = (np.uint32(0), np.uint32(42))
_K1, _K2, _K3 = _np_split(_KEY, 3)
_KA = _np_split(_K1, 2)[1]


def _rotl(x, d):
    return lax.shift_left(x, np.uint32(d)) | lax.shift_right_logical(
        x, np.uint32(32 - d))


def _threefry(ks0, ks1, c2):
    rot = ((13, 15, 26, 6), (17, 29, 16, 24))
    ks2 = ks0 ^ ks1 ^ np.uint32(0x1BD11BDA)
    ks = (ks0, ks1, ks2)
    x0 = ks0 + jnp.zeros_like(c2)
    x1 = c2 + ks1
    for i in range(5):
        for r in rot[i % 2]:
            x0 = x0 + x1
            x1 = _rotl(x1, r)
            x1 = x0 ^ x1
        x0 = x0 + ks[(i + 1) % 3]
        x1 = x1 + ks[(i + 2) % 3] + np.uint32(i + 1)
    return x0 ^ x1


def _gather_q(q_params, labels):
    B = labels.shape[0]
    H = q_params.shape[1]
    info = plsc.get_sparse_core_info()
    nw = info.num_cores * info.num_subcores
    b_per_w = B // nw
    mesh = plsc.VectorSubcoreMesh(core_axis_name="c", subcore_axis_name="s")

    @functools.partial(
        pl.kernel,
        mesh=mesh,
        out_type=jax.ShapeDtypeStruct((B, H), jnp.float32),
        scratch_types=[
            pltpu.VMEM((b_per_w,), jnp.int32),
            pltpu.VMEM((b_per_w, H), jnp.float32),
            pltpu.SemaphoreType.DMA,
        ],
    )
    def k(table_hbm, idx_hbm, out_hbm, idx_v, rows_v, sem):
        wid = lax.axis_index("s") * info.num_cores + lax.axis_index("c")
        base = wid * b_per_w
        pltpu.sync_copy(idx_hbm.at[pl.ds(base, b_per_w)], idx_v)
        pltpu.async_copy(table_hbm.at[idx_v], rows_v, sem).wait()
        pltpu.sync_copy(rows_v, out_hbm.at[pl.ds(base, b_per_w)])

    return k(q_params, labels)


def _sample_body(num_ops, num_scales, max_t, bm,
                 lab_ref, qp_ref, se_ref, nte_ref, augout_ref, scout_ref):
    nts = max_t * num_scales
    g2base = nts
    augbase = nts + max_t
    NEG = np.float32(-1e30)

    i = pl.program_id(0)
    rowg = lax.broadcasted_iota(jnp.uint32, (bm, 128), 0) + np.uint32(bm) * i.astype(jnp.uint32)
    col = lax.broadcasted_iota(jnp.uint32, (bm, 128), 1)
    is_g3 = col < np.uint32(g2base)
    is_g2 = col < np.uint32(augbase)
    counts = jnp.where(
        is_g3, np.uint32(nts) * rowg + col,
        np.uint32(max_t) * rowg + col -
        jnp.where(is_g2, np.uint32(g2base), np.uint32(augbase)))
    k0 = jnp.where(is_g3, np.uint32(_K3[0]),
                   jnp.where(is_g2, np.uint32(_K2[0]), np.uint32(_KA[0])))
    k1 = jnp.where(is_g3, np.uint32(_K3[1]),
                   jnp.where(is_g2, np.uint32(_K2[1]), np.uint32(_KA[1])))
    bits = _threefry(k0, k1, counts)

    fb = lax.shift_right_logical(bits, np.uint32(9)) | np.uint32(0x3F800000)
    f = lax.bitcast_convert_type(fb, jnp.float32) - 1.0
    tiny = np.float32(np.finfo(np.float32).tiny)
    u = jnp.maximum(tiny, f * np.float32(1.0) + tiny)
    gum = -jnp.log(-jnp.log(u))
    augbits = lax.bitcast_convert_type(bits & np.uint32(15), jnp.int32)

    nl = qp_ref.shape[0]
    lab = lab_ref[...]
    colv = lax.broadcasted_iota(jnp.int32, (bm, nl), 1)
    onehot = jnp.where(colv == lab, 1.0, 0.0).astype(jnp.float32)
    q = lax.dot_general(onehot, qp_ref[...], (((1,), (0,)), ((), ())),
                        preferred_element_type=jnp.float32)

    se = se_ref[...]
    w = jnp.concatenate(
        [se, jnp.zeros((num_ops, 32 - num_scales, 128), jnp.float32)],
        axis=1).reshape(num_ops * 32, 128)
    acc = lax.dot_general(q, w, (((1,), (1,)), ((), ())),
                          preferred_element_type=jnp.float32)
    ant = lax.dot_general(q, nte_ref[...], (((1,), (1,)), ((), ())),
                          preferred_element_type=jnp.float32)

    a0 = ant[:, 0:1] + gum[:, g2base:g2base + 1]
    a1 = ant[:, 1:2] + gum[:, g2base + 1:g2base + 2]
    a2 = ant[:, 2:3] + gum[:, g2base + 2:g2base + 3]
    nt_idx = jnp.where(a1 > a0, 1, 0)
    nt_idx = jnp.where(a2 > jnp.maximum(a0, a1), 2, nt_idx)
    n_transforms = nt_idx + 1

    quarters = [acc[:, j * 128:(j + 1) * 128] for j in range(4)]
    col128 = lax.broadcasted_iota(jnp.int32, (bm, 128), 1)
    aug_cols = []
    sc_cols = []
    for t in range(max_t):
        ind_t = augbits[:, augbase + t:augbase + t + 1]
        j0 = lax.shift_right_logical(ind_t, 2)
        lb = ind_t & 3
        fq = jnp.where(j0 == 0, quarters[0],
                       jnp.where(j0 == 1, quarters[1],
                                 jnp.where(j0 == 2, quarters[2], quarters[3])))
        m = jnp.roll(gum, -num_scales * t, axis=1) if t else gum
        m = jnp.where(col128 < num_scales, m, NEG)
        m = jnp.where(col128 >= 32, jnp.roll(m, 32, axis=1), m)
        m = jnp.where(col128 >= 64, jnp.roll(m, 64, axis=1), m)
        vals = jnp.where(lax.shift_right_logical(col128, 5) == lb,
                         fq + m, NEG)
        mx = jnp.max(vals, axis=1, keepdims=True)
        sc = jnp.min(jnp.where(vals == mx, col128, 128), axis=1,
                     keepdims=True)
        sc_cols.append(sc & 31)
        aug_cols.append(jnp.where(t < n_transforms, ind_t, 0))
    augout_ref[...] = jnp.concatenate(aug_cols, axis=1)
    scout_ref[...] = jnp.concatenate(sc_cols, axis=1)


def kernel(imgs, labels, q_params, op_embs, num_transforms_embs, scale_embs):
    B = imgs.shape[0]
    num_ops, num_scales, H = scale_embs.shape
    max_t = num_transforms_embs.shape[0]
    nl = q_params.shape[0]
    lab2 = labels.astype(jnp.int32).reshape(B, 1)

    bm = 1024
    grid = (B // bm,)
    augout, scout = pl.pallas_call(
        functools.partial(_sample_body, num_ops, num_scales, max_t, bm),
        grid=grid,
        in_specs=[
            pl.BlockSpec((bm, 1), lambda i: (i, 0)),
            pl.BlockSpec((nl, H), lambda i: (0, 0)),
            pl.BlockSpec((num_ops, num_scales, H), lambda i: (0, 0, 0)),
            pl.BlockSpec((max_t, H), lambda i: (0, 0)),
        ],
        out_specs=[
            pl.BlockSpec((bm, max_t), lambda i: (i, 0)),
            pl.BlockSpec((bm, max_t), lambda i: (i, 0)),
        ],
        out_shape=[
            jax.ShapeDtypeStruct((B, max_t), jnp.int32),
            jax.ShapeDtypeStruct((B, max_t), jnp.int32),
        ],
    )(lab2, q_params, scale_embs, num_transforms_embs)

    return (augout, scout)
